# Initial kernel scaffold; baseline (speedup 1.0000x reference)
import jax
import jax.numpy as jnp
from jax.experimental import pallas as pl


def kernel(iitem, oitems, Wi, Wo):
    raise NotImplementedError("write your pallas kernel here")


# speedup vs baseline: 3.0227x; 3.0227x over previous
import functools

import jax
import jax.numpy as jnp
from jax import lax
from jax.experimental import pallas as pl
from jax.experimental.pallas import tpu as pltpu
from jax.experimental.pallas import tpu_sc as plsc

ITEM_NUM = 1000000
EMBED = 32
NEGS_PER_CTX = 20
NC, NS, LANES = 2, 16, 16
NW = NC * NS
CHUNK = 112
NCHUNK = 4
PADC = CHUNK * NCHUNK
GPC = CHUNK // LANES


def _build_items(oitems):
    B, C = oitems.shape
    n_neg = C * NEGS_PER_CTX

    def body(o_ref, out_ref):
        pltpu.prng_seed(12345)
        bits = pltpu.prng_random_bits((B, n_neg))
        bits = lax.bitcast_convert_type(bits, jnp.int32)
        negs = (bits & jnp.int32(0x7FFFFFFF)) % jnp.int32(ITEM_NUM)
        pad = jnp.zeros((B, PADC - C - n_neg), jnp.int32)
        out_ref[...] = jnp.concatenate([o_ref[...], negs, pad], axis=1)

    return pl.pallas_call(
        body, out_shape=jax.ShapeDtypeStruct((B, PADC), jnp.int32)
    )(oitems.astype(jnp.int32))


def _sc_scores(items3, iitem, Wi, Wo):
    B = iitem.shape[0]
    BPW = B // NW

    mesh = plsc.VectorSubcoreMesh(
        core_axis_name="c", subcore_axis_name="s",
        num_cores=NC, num_subcores=NS,
    )

    @functools.partial(
        pl.kernel,
        out_type=jax.ShapeDtypeStruct((B, PADC), jnp.float32),
        mesh=mesh,
        compiler_params=pltpu.CompilerParams(
            needs_layout_passes=False, use_tc_tiling_on_sc=False
        ),
        scratch_types=[
            pltpu.VMEM((BPW,), jnp.int32),
            pltpu.VMEM((BPW, EMBED), jnp.float32),
            pltpu.VMEM((NCHUNK, CHUNK), jnp.int32),
            pltpu.VMEM((NCHUNK, CHUNK, EMBED), jnp.float32),
            pltpu.VMEM((PADC,), jnp.float32),
            pltpu.SemaphoreType.DMA,
        ],
    )
    def sc_kernel(items_hbm, iitem_hbm, wi_hbm, wo_hbm, out_hbm,
                  iidx_v, ivecs_v, idx_v, rows_v, scores_v, gsem):
        wid = lax.axis_index("s") * NC + lax.axis_index("c")
        base = wid * BPW
        pltpu.sync_copy(iitem_hbm.at[pl.ds(base, BPW)], iidx_v)
        pltpu.async_copy(wi_hbm.at[iidx_v], ivecs_v, gsem).wait()

        def b_body(lb, carry):
            b = base + lb
            pltpu.sync_copy(items_hbm.at[b], idx_v)
            cps = [
                pltpu.async_copy(wo_hbm.at[idx_v.at[j]], rows_v.at[j], gsem)
                for j in range(NCHUNK)
            ]
            for cp in cps:
                cp.wait()
            iv0 = ivecs_v[lb, pl.ds(0, LANES)]
            iv1 = ivecs_v[lb, pl.ds(LANES, LANES)]
            ivs = [iv0[d] for d in range(LANES)] + [iv1[d] for d in range(LANES)]
            for j in range(NCHUNK):
                jv = jnp.full((LANES,), j, jnp.int32)

                def g_body(gi, c2, jv=jv, j=j):
                    rv = gi * LANES + lax.iota(jnp.int32, LANES)
                    acc = [jnp.zeros((LANES,), jnp.float32) for _ in range(4)]
                    for d in range(EMBED):
                        cv = jnp.full((LANES,), d, jnp.int32)
                        vals = plsc.load_gather(rows_v, [jv, rv, cv])
                        acc[d % 4] = acc[d % 4] + vals * ivs[d]
                    s = (acc[0] + acc[1]) + (acc[2] + acc[3])
                    off = j * CHUNK + gi * LANES
                    scores_v[pl.ds(off, LANES)] = s
                    return c2

                lax.fori_loop(0, GPC, g_body, 0)
            pltpu.sync_copy(scores_v, out_hbm.at[b])
            return carry

        lax.fori_loop(0, BPW, b_body, 0)

    return sc_kernel(items3, iitem, Wi, Wo)


def _tc_loss(scores, C):
    n_cols = C + C * NEGS_PER_CTX

    def body(s_ref, out_ref):
        s = s_ref[...]
        col = lax.broadcasted_iota(jnp.int32, s.shape, 1)
        x = jnp.where(col < C, s, -s)
        ls = jnp.minimum(x, 0.0) - jnp.log1p(jnp.exp(-jnp.abs(x)))
        ls = jnp.where(col < n_cols, ls, 0.0)
        per_b = jnp.sum(ls, axis=1) / C
        out_ref[...] = (-jnp.mean(per_b))[None, None]

    out = pl.pallas_call(
        body, out_shape=jax.ShapeDtypeStruct((1, 1), jnp.float32)
    )(scores)
    return out[0, 0]


def kernel(iitem, oitems, Wi, Wo):
    B, C = oitems.shape
    items = _build_items(oitems)
    items3 = items.reshape(B, NCHUNK, CHUNK)
    scores = _sc_scores(items3, iitem.astype(jnp.int32), Wi, Wo)
    return _tc_loss(scores, C)
